# Initial kernel scaffold; baseline (speedup 1.0000x reference)
import jax
import jax.numpy as jnp
from jax.experimental import pallas as pl


def kernel(x_one, edge_index_one, graph_ids_one, x_two, edge_index_two, graph_ids_two, tgt, W_emb, W_g1_0, W_g1_1, W_g2_0, W_g2_1, W_ff, b_ff):
    raise NotImplementedError("write your pallas kernel here")


# speedup vs baseline: 2.7359x; 2.7359x over previous
import functools

import jax
import jax.numpy as jnp
from jax import lax
from jax.experimental import pallas as pl
from jax.experimental.pallas import tpu as pltpu
from jax.experimental.pallas import tpu_sc as plsc

N = 10000
E = 320000
D = 128
B = 32
C = 2

NSUB = 16
NPAD = 10240
SLAB = NPAD // NSUB
CHUNK = 128
CH = 160
IG = 8
EPAD = NSUB * CH * CHUNK

_mesh = plsc.VectorSubcoreMesh(core_axis_name="c", subcore_axis_name="s")


@functools.partial(
    pl.kernel,
    mesh=_mesh,
    out_type=(
        jax.ShapeDtypeStruct((2, NSUB, SLAB, D), jnp.float32),
        jax.ShapeDtypeStruct((2, NSUB, SLAB, 16), jnp.float32),
    ),
    scratch_types=[
        pltpu.VMEM((IG, CHUNK), jnp.int32),
        pltpu.VMEM((IG, CHUNK), jnp.int32),
        pltpu.VMEM((CHUNK, D), jnp.float32),
        pltpu.VMEM((CHUNK, 16), jnp.float32),
        pltpu.VMEM_SHARED((NPAD, D), jnp.float32),
        pltpu.VMEM_SHARED((NPAD, 16), jnp.float32),
        pltpu.SemaphoreType.DMA,
    ],
)
def _sc_agg_deg(h_hbm, src_hbm, dst_hbm, z_hbm, z16_hbm, ones_hbm,
                agg_out, deg_out,
                src_v, dst_v, rows_v, ones_v, acc_sh, deg_sh, sem):
    c = lax.axis_index("c")
    s = lax.axis_index("s")
    r0 = s * SLAB
    pltpu.sync_copy(z_hbm.at[pl.ds(r0, SLAB)], acc_sh.at[pl.ds(r0, SLAB)])
    pltpu.sync_copy(z16_hbm.at[pl.ds(r0, SLAB)], deg_sh.at[pl.ds(r0, SLAB)])
    pltpu.sync_copy(ones_hbm, ones_v)
    plsc.subcore_barrier()

    def group(g, carry):
        pltpu.sync_copy(src_hbm.at[c, s, pl.ds(g * IG, IG)], src_v)
        pltpu.sync_copy(dst_hbm.at[c, s, pl.ds(g * IG, IG)], dst_v)
        return carry

    lax.fori_loop(0, CH // IG, group, 0)
    plsc.subcore_barrier()
    pltpu.sync_copy(acc_sh.at[pl.ds(r0, SLAB)], agg_out.at[c, s])
    pltpu.sync_copy(deg_sh.at[pl.ds(r0, SLAB)], deg_out.at[c, s])


@functools.partial(
    pl.kernel,
    mesh=_mesh,
    out_type=(jax.ShapeDtypeStruct((2, NSUB, SLAB, D), jnp.float32),
              jax.ShapeDtypeStruct((2, NSUB, SLAB, D), jnp.float32)),
    scratch_types=[
        pltpu.VMEM((IG, CHUNK), jnp.int32),
        pltpu.VMEM((IG, CHUNK), jnp.int32),
        pltpu.VMEM((CHUNK, D), jnp.float32),
        pltpu.VMEM((CHUNK, D), jnp.float32),
        pltpu.VMEM_SHARED((NPAD, D), jnp.float32),
        pltpu.SemaphoreType.DMA,
    ],
)
def _sc_agg_deg(h_hbm, src_hbm, dst_hbm, z_hbm, ones_hbm,
                agg_out, deg_out,
                src_v, dst_v, rows_v, ones_v, acc_sh, sem):
    c = lax.axis_index("c")
    s = lax.axis_index("s")
    r0 = s * SLAB
    pltpu.sync_copy(z_hbm.at[pl.ds(r0, SLAB)], acc_sh.at[pl.ds(r0, SLAB)])
    pltpu.sync_copy(ones_hbm, ones_v)
    plsc.subcore_barrier()

    def dgroup(g, carry):
        pltpu.sync_copy(dst_hbm.at[c, s, pl.ds(g * IG, IG)], dst_v)

        def dstep(j, cc):
            pltpu.sync_copy(ones_v, acc_sh.at[dst_v.at[j]], add=True)
            return cc

        return lax.fori_loop(0, IG, dstep, carry)

    lax.fori_loop(0, CH // IG, dgroup, 0)
    plsc.subcore_barrier()
    pltpu.sync_copy(acc_sh.at[pl.ds(r0, SLAB)], deg_out.at[c, s])
    plsc.subcore_barrier()
    pltpu.sync_copy(z_hbm.at[pl.ds(r0, SLAB)], acc_sh.at[pl.ds(r0, SLAB)])
    plsc.subcore_barrier()

    def group(g, carry):
        pltpu.sync_copy(src_hbm.at[c, s, pl.ds(g * IG, IG)], src_v)
        pltpu.sync_copy(dst_hbm.at[c, s, pl.ds(g * IG, IG)], dst_v)

        def step(j, cc):
            pltpu.async_copy(h_hbm.at[src_v.at[j]], rows_v, sem).wait()
            pltpu.sync_copy(rows_v, acc_sh.at[dst_v.at[j]], add=True)
            return cc

        return lax.fori_loop(0, IG, step, carry)

    lax.fori_loop(0, CH // IG, group, 0)
    plsc.subcore_barrier()
    pltpu.sync_copy(acc_sh.at[pl.ds(r0, SLAB)], agg_out.at[c, s])


@functools.partial(
    pl.kernel,
    mesh=_mesh,
    out_type=jax.ShapeDtypeStruct((2, NSUB, SLAB, D), jnp.float32),
    scratch_types=[
        pltpu.VMEM((IG, CHUNK), jnp.int32),
        pltpu.VMEM((IG, CHUNK), jnp.int32),
        pltpu.VMEM((CHUNK, D), jnp.float32),
        pltpu.VMEM_SHARED((NPAD, D), jnp.float32),
        pltpu.SemaphoreType.DMA,
    ],
)
def _sc_agg(h_hbm, src_hbm, dst_hbm, z_hbm,
            agg_out,
            src_v, dst_v, rows_v, acc_sh, sem):
    c = lax.axis_index("c")
    s = lax.axis_index("s")
    r0 = s * SLAB
    pltpu.sync_copy(z_hbm.at[pl.ds(r0, SLAB)], acc_sh.at[pl.ds(r0, SLAB)])
    plsc.subcore_barrier()

    def group(g, carry):
        pltpu.sync_copy(src_hbm.at[c, s, pl.ds(g * IG, IG)], src_v)
        pltpu.sync_copy(dst_hbm.at[c, s, pl.ds(g * IG, IG)], dst_v)

        def step(j, cc):
            pltpu.async_copy(h_hbm.at[src_v.at[j]], rows_v, sem).wait()
            pltpu.sync_copy(rows_v, acc_sh.at[dst_v.at[j]], add=True)
            return cc

        return lax.fori_loop(0, IG, step, carry)

    lax.fori_loop(0, CH // IG, group, 0)
    plsc.subcore_barrier()
    pltpu.sync_copy(acc_sh.at[pl.ds(r0, SLAB)], agg_out.at[c, s])


def _emb_body(x_ref, w_ref, o_ref):
    o_ref[...] = jnp.dot(x_ref[...], w_ref[...], preferred_element_type=jnp.float32)


def _layer_body(agg_ref, deg_ref, w_ref, o_ref):
    d = jnp.maximum(deg_ref[:, 0:1], 1.0)
    a = agg_ref[...] / d
    o_ref[...] = jnp.maximum(
        jnp.dot(a, w_ref[0], preferred_element_type=jnp.float32), 0.0)


def _layer_readout_body(agg_ref, deg_ref, w_ref, gid_ref, sums_ref, cnts_ref):
    i = pl.program_id(0)
    d = jnp.maximum(deg_ref[:, 0:1], 1.0)
    a = agg_ref[...] / d
    h = jnp.maximum(jnp.dot(a, w_ref[0], preferred_element_type=jnp.float32), 0.0)
    gid = gid_ref[0]
    ohT = (lax.broadcasted_iota(jnp.int32, (B, CHUNK), 0) == gid).astype(jnp.float32)
    part = jnp.dot(ohT, h, preferred_element_type=jnp.float32)
    cnt = jnp.broadcast_to(jnp.sum(ohT, axis=1, keepdims=True), (B, D))

    @pl.when(i % (NPAD // CHUNK) == 0)
    def _init():
        sums_ref[0] = part
        cnts_ref[0] = cnt

    @pl.when(i % (NPAD // CHUNK) != 0)
    def _acc():
        sums_ref[0] += part
        cnts_ref[0] += cnt


def _head_body(sums_ref, cnts_ref, wff_ref, b_ref, toh_ref, sm_ref, loss_ref):
    f1 = sums_ref[0] / jnp.maximum(cnts_ref[0], 1.0)
    f2 = sums_ref[1] / jnp.maximum(cnts_ref[1], 1.0)
    e = (f1 - f2) ** 2
    logits = jnp.dot(e, wff_ref[...], preferred_element_type=jnp.float32) + b_ref[0:1, :]
    l = jnp.where(logits > 0, logits, 0.01 * logits)
    m = jnp.max(l, axis=1, keepdims=True)
    z = l - m
    lse = jnp.log(jnp.sum(jnp.exp(z), axis=1, keepdims=True))
    logp = z - lse
    loss = -jnp.sum(logp * toh_ref[...]) / B
    sm_ref[...] = jnp.exp(logp)
    loss_ref[...] = jnp.full((8, 128), loss)


_ROWS = 2 * NPAD
_GRID = _ROWS // CHUNK


def _emb_matmul(x_flat, w):
    return pl.pallas_call(
        _emb_body,
        grid=(_GRID,),
        in_specs=[pl.BlockSpec((CHUNK, D), lambda i: (i, 0)),
                  pl.BlockSpec((D, D), lambda i: (0, 0))],
        out_specs=pl.BlockSpec((CHUNK, D), lambda i: (i, 0)),
        out_shape=jax.ShapeDtypeStruct((_ROWS, D), jnp.float32),
    )(x_flat, w)


def _layer(agg_flat, deg_flat, w_stack):
    return pl.pallas_call(
        _layer_body,
        grid=(_GRID,),
        in_specs=[pl.BlockSpec((CHUNK, D), lambda i: (i, 0)),
                  pl.BlockSpec((CHUNK, D), lambda i: (i, 0)),
                  pl.BlockSpec((1, D, D), lambda i: (i // (_GRID // 2), 0, 0))],
        out_specs=pl.BlockSpec((CHUNK, D), lambda i: (i, 0)),
        out_shape=jax.ShapeDtypeStruct((_ROWS, D), jnp.float32),
    )(agg_flat, deg_flat, w_stack)


def _layer_readout(agg_flat, deg_flat, w_stack, gids_r):
    side = _GRID // 2
    return pl.pallas_call(
        _layer_readout_body,
        grid=(_GRID,),
        in_specs=[pl.BlockSpec((CHUNK, D), lambda i: (i, 0)),
                  pl.BlockSpec((CHUNK, D), lambda i: (i, 0)),
                  pl.BlockSpec((1, D, D), lambda i: (i // side, 0, 0)),
                  pl.BlockSpec((1, 1, CHUNK), lambda i: (i, 0, 0))],
        out_specs=[pl.BlockSpec((1, B, D), lambda i: (i // side, 0, 0)),
                   pl.BlockSpec((1, B, D), lambda i: (i // side, 0, 0))],
        out_shape=[jax.ShapeDtypeStruct((2, B, D), jnp.float32),
                   jax.ShapeDtypeStruct((2, B, D), jnp.float32)],
    )(agg_flat, deg_flat, w_stack, gids_r)


def _head(sums, cnts, wff, b_pad, toh):
    return pl.pallas_call(
        _head_body,
        in_specs=[pl.BlockSpec((2, B, D), lambda: (0, 0, 0)),
                  pl.BlockSpec((2, B, D), lambda: (0, 0, 0)),
                  pl.BlockSpec((D, C), lambda: (0, 0)),
                  pl.BlockSpec((8, C), lambda: (0, 0)),
                  pl.BlockSpec((B, C), lambda: (0, 0))],
        out_specs=[pl.BlockSpec((B, C), lambda: (0, 0)),
                   pl.BlockSpec((8, 128), lambda: (0, 0))],
        out_shape=[jax.ShapeDtypeStruct((B, C), jnp.float32),
                   jax.ShapeDtypeStruct((8, 128), jnp.float32)],
    )(sums, cnts, wff, b_pad, toh)


def _prep_edges(edge_index, side):
    src = edge_index[0]
    dst = edge_index[1]
    pad = EPAD - E
    src_g = jnp.concatenate(
        [src + side * NPAD, jnp.full((pad,), side * NPAD, jnp.int32)])
    dst_l = jnp.concatenate([dst, jnp.full((pad,), N, jnp.int32)])
    return src_g.reshape(NSUB, CH, CHUNK), dst_l.reshape(NSUB, CH, CHUNK)


def kernel(x_one, edge_index_one, graph_ids_one, x_two, edge_index_two,
           graph_ids_two, tgt, W_emb, W_g1_0, W_g1_1, W_g2_0, W_g2_1, W_ff, b_ff):
    x_flat = (jnp.zeros((2, NPAD, D), jnp.float32)
              .at[0, :N].set(x_one)
              .at[1, :N].set(x_two)
              .reshape(_ROWS, D))
    s1, d1 = _prep_edges(edge_index_one, 0)
    s2, d2 = _prep_edges(edge_index_two, 1)
    src_all = jnp.stack([s1, s2])
    dst_all = jnp.stack([d1, d2])
    gids_r = (jnp.full((2, NPAD), B, jnp.int32)
              .at[0, :N].set(graph_ids_one)
              .at[1, :N].set(graph_ids_two)
              .reshape(_GRID, 1, CHUNK))
    zeros = jnp.zeros((NPAD, D), jnp.float32)
    ones128 = jnp.ones((CHUNK, D), jnp.float32)
    w_stack1 = jnp.stack([W_g1_0, W_g2_0])
    w_stack2 = jnp.stack([W_g1_1, W_g2_1])
    b_pad = jnp.broadcast_to(b_ff[None, :], (8, C))
    toh = (tgt[:, None] == jnp.arange(C, dtype=jnp.int32)[None, :]).astype(jnp.float32)

    h0 = _emb_matmul(x_flat, W_emb)
    agg1_sc, deg_sc = _sc_agg_deg(h0, src_all, dst_all, zeros, ones128)
    agg1_flat = agg1_sc.reshape(_ROWS, D)
    deg_flat = deg_sc.reshape(_ROWS, D)
    h1 = _layer(agg1_flat, deg_flat, w_stack1)
    agg2_flat = _sc_agg(h1, src_all, dst_all, zeros).reshape(_ROWS, D)
    sums, cnts = _layer_readout(agg2_flat, deg_flat, w_stack2, gids_r)
    sm, loss = _head(sums, cnts, W_ff, b_pad, toh)
    return sm, loss[0, 0]


# speedup vs baseline: 3.6222x; 1.3240x over previous
import functools

import jax
import jax.numpy as jnp
from jax import lax
from jax.experimental import pallas as pl
from jax.experimental.pallas import tpu as pltpu
from jax.experimental.pallas import tpu_sc as plsc

N = 10000
E = 320000
D = 128
B = 32
C = 2

NSUB = 16
NPAD = 10240
SLAB = NPAD // NSUB
CHUNK = 128
CH = 160
IG = 8
NG = CH // IG
EPAD = NSUB * CH * CHUNK

_mesh = plsc.VectorSubcoreMesh(core_axis_name="c", subcore_axis_name="s")


def _fill(buf, val):
    def row(i, carry):
        def col(k, carry2):
            buf[i, pl.ds(pl.multiple_of(k * 16, 16), 16)] = jnp.full(
                (16,), val, jnp.float32)
            return carry2
        return lax.fori_loop(0, D // 16, col, carry)
    lax.fori_loop(0, CHUNK, row, 0)


def _zero_slab(buf, acc_sh, r0):
    for t in range(SLAB // CHUNK):
        pltpu.sync_copy(buf, acc_sh.at[pl.ds(r0 + t * CHUNK, CHUNK)])


def _agg_loop(h_hbm, src_hbm, dst_hbm, acc_sh, src_v, dst_v, buf_a, buf_b,
              sem_a, sem_b, c, s):
    def body(g, carry):
        pltpu.sync_copy(src_hbm.at[c, s, pl.ds(g * IG, IG)], src_v)
        pltpu.sync_copy(dst_hbm.at[c, s, pl.ds(g * IG, IG)], dst_v)
        pending = pltpu.async_copy(h_hbm.at[src_v.at[0]], buf_a, sem_a)
        for j in range(IG):
            cur_buf = buf_a if j % 2 == 0 else buf_b
            nxt_buf = buf_b if j % 2 == 0 else buf_a
            nxt_sem = sem_b if j % 2 == 0 else sem_a
            nxt = None
            if j < IG - 1:
                nxt = pltpu.async_copy(
                    h_hbm.at[src_v.at[j + 1]], nxt_buf, nxt_sem)
            pending.wait()
            pltpu.sync_copy(cur_buf, acc_sh.at[dst_v.at[j]], add=True)
            pending = nxt
        return carry

    lax.fori_loop(0, NG, body, 0)


def _deg_loop(dst_hbm, acc_sh, dst_v, ones_buf, sem, c, s):
    def body(g, carry):
        pltpu.sync_copy(dst_hbm.at[c, s, pl.ds(g * IG, IG)], dst_v)
        handles = [
            pltpu.async_copy(ones_buf, acc_sh.at[dst_v.at[j]], sem, add=True)
            for j in range(IG)
        ]
        for h in handles:
            h.wait()
        return carry

    lax.fori_loop(0, NG, body, 0)


@functools.partial(
    pl.kernel,
    mesh=_mesh,
    out_type=(
        jax.ShapeDtypeStruct((2, NSUB, SLAB, D), jnp.float32),
        jax.ShapeDtypeStruct((2, NSUB, SLAB, D), jnp.float32),
    ),
    scratch_types=[
        pltpu.VMEM((IG, CHUNK), jnp.int32),
        pltpu.VMEM((IG, CHUNK), jnp.int32),
        pltpu.VMEM((CHUNK, D), jnp.float32),
        pltpu.VMEM((CHUNK, D), jnp.float32),
        pltpu.VMEM_SHARED((NPAD, D), jnp.float32),
        pltpu.SemaphoreType.DMA,
        pltpu.SemaphoreType.DMA,
    ],
)
def _sc_agg_deg(h_hbm, src_hbm, dst_hbm,
                agg_out, deg_out,
                src_v, dst_v, buf_a, buf_b, acc_sh, sem_a, sem_b):
    c = lax.axis_index("c")
    s = lax.axis_index("s")
    r0 = s * SLAB
    _fill(buf_a, 0.0)
    _zero_slab(buf_a, acc_sh, r0)
    _fill(buf_b, 1.0)
    plsc.subcore_barrier()
    _deg_loop(dst_hbm, acc_sh, dst_v, buf_b, sem_a, c, s)
    plsc.subcore_barrier()
    pltpu.sync_copy(acc_sh.at[pl.ds(r0, SLAB)], deg_out.at[c, s])
    plsc.subcore_barrier()
    _fill(buf_a, 0.0)
    _zero_slab(buf_a, acc_sh, r0)
    plsc.subcore_barrier()
    _agg_loop(h_hbm, src_hbm, dst_hbm, acc_sh, src_v, dst_v, buf_a, buf_b,
              sem_a, sem_b, c, s)
    plsc.subcore_barrier()
    pltpu.sync_copy(acc_sh.at[pl.ds(r0, SLAB)], agg_out.at[c, s])


@functools.partial(
    pl.kernel,
    mesh=_mesh,
    out_type=jax.ShapeDtypeStruct((2, NSUB, SLAB, D), jnp.float32),
    scratch_types=[
        pltpu.VMEM((IG, CHUNK), jnp.int32),
        pltpu.VMEM((IG, CHUNK), jnp.int32),
        pltpu.VMEM((CHUNK, D), jnp.float32),
        pltpu.VMEM((CHUNK, D), jnp.float32),
        pltpu.VMEM_SHARED((NPAD, D), jnp.float32),
        pltpu.SemaphoreType.DMA,
        pltpu.SemaphoreType.DMA,
    ],
)
def _sc_agg(h_hbm, src_hbm, dst_hbm,
            agg_out,
            src_v, dst_v, buf_a, buf_b, acc_sh, sem_a, sem_b):
    c = lax.axis_index("c")
    s = lax.axis_index("s")
    r0 = s * SLAB
    _fill(buf_a, 0.0)
    _zero_slab(buf_a, acc_sh, r0)
    plsc.subcore_barrier()
    _agg_loop(h_hbm, src_hbm, dst_hbm, acc_sh, src_v, dst_v, buf_a, buf_b,
              sem_a, sem_b, c, s)
    plsc.subcore_barrier()
    pltpu.sync_copy(acc_sh.at[pl.ds(r0, SLAB)], agg_out.at[c, s])


def _emb_body(x_ref, w_ref, o_ref):
    o_ref[...] = jnp.dot(x_ref[...], w_ref[...], preferred_element_type=jnp.float32)


def _layer_body(agg_ref, deg_ref, w_ref, o_ref):
    d = jnp.maximum(deg_ref[:, 0:1], 1.0)
    a = agg_ref[...] / d
    o_ref[...] = jnp.maximum(
        jnp.dot(a, w_ref[0], preferred_element_type=jnp.float32), 0.0)


RB = 512
_ROWS = 2 * NPAD
_GRID = _ROWS // RB


def _layer_readout_body(agg_ref, deg_ref, w_ref, gid_ref, sums_ref, cnts_ref):
    i = pl.program_id(0)
    d = jnp.maximum(deg_ref[:, 0:1], 1.0)
    a = agg_ref[...] / d
    h = jnp.maximum(jnp.dot(a, w_ref[0], preferred_element_type=jnp.float32), 0.0)
    gid = gid_ref[0]
    ohT = (lax.broadcasted_iota(jnp.int32, (B, RB), 0) == gid).astype(jnp.float32)
    part = jnp.dot(ohT, h, preferred_element_type=jnp.float32)
    cnt = jnp.broadcast_to(jnp.sum(ohT, axis=1, keepdims=True), (B, D))

    @pl.when(i % (_GRID // 2) == 0)
    def _init():
        sums_ref[0] = part
        cnts_ref[0] = cnt

    @pl.when(i % (_GRID // 2) != 0)
    def _acc():
        sums_ref[0] += part
        cnts_ref[0] += cnt


def _head_body(sums_ref, cnts_ref, wff_ref, b_ref, toh_ref, sm_ref, loss_ref):
    f1 = sums_ref[0] / jnp.maximum(cnts_ref[0], 1.0)
    f2 = sums_ref[1] / jnp.maximum(cnts_ref[1], 1.0)
    e = (f1 - f2) ** 2
    logits = jnp.dot(e, wff_ref[...], preferred_element_type=jnp.float32) + b_ref[0:1, :]
    l = jnp.where(logits > 0, logits, 0.01 * logits)
    m = jnp.max(l, axis=1, keepdims=True)
    z = l - m
    lse = jnp.log(jnp.sum(jnp.exp(z), axis=1, keepdims=True))
    logp = z - lse
    loss = -jnp.sum(logp * toh_ref[...]) / B
    sm_ref[...] = jnp.exp(logp)
    loss_ref[...] = jnp.full((8, 128), loss)


def _emb_matmul(x_flat, w):
    return pl.pallas_call(
        _emb_body,
        grid=(_GRID,),
        in_specs=[pl.BlockSpec((RB, D), lambda i: (i, 0)),
                  pl.BlockSpec((D, D), lambda i: (0, 0))],
        out_specs=pl.BlockSpec((RB, D), lambda i: (i, 0)),
        out_shape=jax.ShapeDtypeStruct((_ROWS, D), jnp.float32),
    )(x_flat, w)


def _layer(agg_flat, deg_flat, w_stack):
    return pl.pallas_call(
        _layer_body,
        grid=(_GRID,),
        in_specs=[pl.BlockSpec((RB, D), lambda i: (i, 0)),
                  pl.BlockSpec((RB, D), lambda i: (i, 0)),
                  pl.BlockSpec((1, D, D), lambda i: (i // (_GRID // 2), 0, 0))],
        out_specs=pl.BlockSpec((RB, D), lambda i: (i, 0)),
        out_shape=jax.ShapeDtypeStruct((_ROWS, D), jnp.float32),
    )(agg_flat, deg_flat, w_stack)


def _layer_readout(agg_flat, deg_flat, w_stack, gids_r):
    side = _GRID // 2
    return pl.pallas_call(
        _layer_readout_body,
        grid=(_GRID,),
        in_specs=[pl.BlockSpec((RB, D), lambda i: (i, 0)),
                  pl.BlockSpec((RB, D), lambda i: (i, 0)),
                  pl.BlockSpec((1, D, D), lambda i: (i // side, 0, 0)),
                  pl.BlockSpec((1, 1, RB), lambda i: (i, 0, 0))],
        out_specs=[pl.BlockSpec((1, B, D), lambda i: (i // side, 0, 0)),
                   pl.BlockSpec((1, B, D), lambda i: (i // side, 0, 0))],
        out_shape=[jax.ShapeDtypeStruct((2, B, D), jnp.float32),
                   jax.ShapeDtypeStruct((2, B, D), jnp.float32)],
    )(agg_flat, deg_flat, w_stack, gids_r)


def _head(sums, cnts, wff, b_pad, toh):
    return pl.pallas_call(
        _head_body,
        in_specs=[pl.BlockSpec((2, B, D), lambda: (0, 0, 0)),
                  pl.BlockSpec((2, B, D), lambda: (0, 0, 0)),
                  pl.BlockSpec((D, C), lambda: (0, 0)),
                  pl.BlockSpec((8, C), lambda: (0, 0)),
                  pl.BlockSpec((B, C), lambda: (0, 0))],
        out_specs=[pl.BlockSpec((B, C), lambda: (0, 0)),
                   pl.BlockSpec((8, 128), lambda: (0, 0))],
        out_shape=[jax.ShapeDtypeStruct((B, C), jnp.float32),
                   jax.ShapeDtypeStruct((8, 128), jnp.float32)],
    )(sums, cnts, wff, b_pad, toh)


def _prep_edges(edge_index, side):
    src = edge_index[0]
    dst = edge_index[1]
    pad = EPAD - E
    src_g = jnp.concatenate(
        [src + side * NPAD, jnp.full((pad,), side * NPAD, jnp.int32)])
    dst_l = jnp.concatenate([dst, jnp.full((pad,), N, jnp.int32)])
    return src_g.reshape(NSUB, CH, CHUNK), dst_l.reshape(NSUB, CH, CHUNK)


def kernel(x_one, edge_index_one, graph_ids_one, x_two, edge_index_two,
           graph_ids_two, tgt, W_emb, W_g1_0, W_g1_1, W_g2_0, W_g2_1, W_ff, b_ff):
    x_flat = (jnp.zeros((2, NPAD, D), jnp.float32)
              .at[0, :N].set(x_one)
              .at[1, :N].set(x_two)
              .reshape(_ROWS, D))
    s1, d1 = _prep_edges(edge_index_one, 0)
    s2, d2 = _prep_edges(edge_index_two, 1)
    src_all = jnp.stack([s1, s2])
    dst_all = jnp.stack([d1, d2])
    gids_r = (jnp.full((2, NPAD), B, jnp.int32)
              .at[0, :N].set(graph_ids_one)
              .at[1, :N].set(graph_ids_two)
              .reshape(_ROWS // RB, 1, RB))
    w_stack1 = jnp.stack([W_g1_0, W_g2_0])
    w_stack2 = jnp.stack([W_g1_1, W_g2_1])
    b_pad = jnp.broadcast_to(b_ff[None, :], (8, C))
    toh = (tgt[:, None] == jnp.arange(C, dtype=jnp.int32)[None, :]).astype(jnp.float32)

    h0 = _emb_matmul(x_flat, W_emb)
    agg1_sc, deg_sc = _sc_agg_deg(h0, src_all, dst_all)
    agg1_flat = agg1_sc.reshape(_ROWS, D)
    deg_flat = deg_sc.reshape(_ROWS, D)
    h1 = _layer(agg1_flat, deg_flat, w_stack1)
    agg2_flat = _sc_agg(h1, src_all, dst_all).reshape(_ROWS, D)
    sums, cnts = _layer_readout(agg2_flat, deg_flat, w_stack2, gids_r)
    sm, loss = _head(sums, cnts, W_ff, b_pad, toh)
    return sm, loss[0, 0]
